# Initial kernel scaffold; baseline (speedup 1.0000x reference)
import jax
import jax.numpy as jnp
from jax.experimental import pallas as pl


def kernel(idx, date, nns, train_dates, mem, W_pos, b_pos, W_field, b_field, W_adapt, b_adapt):
    raise NotImplementedError("write your pallas kernel here")


# speedup vs baseline: 2.2109x; 2.2109x over previous
import functools

import jax
import jax.numpy as jnp
from jax import lax
from jax.experimental import pallas as pl
from jax.experimental.pallas import tpu as pltpu
from jax.experimental.pallas import tpu_sc as plsc

_LANES = 16
_CHUNK = 128


def _make_sc_gather(B, K, F, C, NC, NS):
    NW = NC * NS
    bpw = B // NW
    mesh = plsc.VectorSubcoreMesh(core_axis_name="c", subcore_axis_name="s")

    @functools.partial(
        pl.kernel,
        mesh=mesh,
        compiler_params=pltpu.CompilerParams(use_tc_tiling_on_sc=False),
        out_type=[
            jax.ShapeDtypeStruct((K, B, F), jnp.float32),
            jax.ShapeDtypeStruct((K, B, C), jnp.float32),
        ],
        scratch_types=[
            pltpu.VMEM((bpw,), jnp.int32),
            pltpu.VMEM((bpw,), jnp.int32),
            pltpu.VMEM((bpw,), jnp.int32),
            pltpu.VMEM((bpw, F), jnp.float32),
            pltpu.VMEM((bpw, C), jnp.float32),
            pltpu.SemaphoreType.DMA,
        ],
    )
    def sc_gather(idx_hbm, nnsf_hbm, td_hbm, mem_hbm, td_out, mem_out,
                  idx_v, fidx_v, col_v, tdb, memb, sem):
        wid = lax.axis_index("s") * NC + lax.axis_index("c")
        base = wid * bpw
        pltpu.sync_copy(idx_hbm.at[pl.ds(base, bpw)], idx_v)
        for k in range(K):
            def ext(i, _):
                v = idx_v[pl.ds(i * _LANES, _LANES)]
                fidx_v[pl.ds(i * _LANES, _LANES)] = v * K + k
                return 0
            lax.fori_loop(0, bpw // _LANES, ext, 0)
            for j in range(bpw // _CHUNK):
                sl = pl.ds(j * _CHUNK, _CHUNK)
                pltpu.async_copy(nnsf_hbm.at[fidx_v.at[sl]], col_v.at[sl],
                                 sem).wait()
                pltpu.async_copy(td_hbm.at[col_v.at[sl]], tdb.at[sl], sem).wait()
                pltpu.async_copy(mem_hbm.at[col_v.at[sl]], memb.at[sl], sem).wait()
            pltpu.sync_copy(tdb, td_out.at[k, pl.ds(base, bpw)])
            pltpu.sync_copy(memb, mem_out.at[k, pl.ds(base, bpw)])

    return sc_gather


def _tc_body(K, HID, date_ref, td_ref, mem_ref, wp_ref, bp_ref, wf_ref,
             bf_ref, wa_ref, ba_ref, out_ref):
    date = date_ref[...]
    wp = wp_ref[...]
    bp = bp_ref[...]
    pos_q = jnp.dot(date, wp, preferred_element_type=jnp.float32) + bp
    field = jnp.dot(date, wf_ref[...], preferred_element_type=jnp.float32) + bf_ref[...]

    diffs, negs = [], []
    for k in range(K):
        pos_r = jnp.dot(td_ref[k], wp, preferred_element_type=jnp.float32) + bp
        d = pos_q - pos_r
        diffs.append(d)
        negs.append(-jnp.sqrt(jnp.sum(d * d, axis=1, keepdims=True)))
    m = negs[0]
    for k in range(1, K):
        m = jnp.maximum(m, negs[k])
    es = [jnp.exp(n - m) for n in negs]
    inv = 1.0 / sum(es)
    wdiff = sum(es[k] * diffs[k] for k in range(K)) * inv
    wmem = sum(es[k] * mem_ref[k] for k in range(K)) * inv

    out = jnp.dot(wmem, wa_ref[...], preferred_element_type=jnp.float32) + ba_ref[...]
    nc = out.shape[1]
    for h in range(HID):
        out += wdiff[:, h:h + 1] * field[:, h * nc:(h + 1) * nc]
    out_ref[...] = out


def kernel(idx, date, nns, train_dates, mem, W_pos, b_pos, W_field, b_field,
           W_adapt, b_adapt):
    B = idx.shape[0]
    K = nns.shape[1]
    F = train_dates.shape[1]
    C = mem.shape[1]
    HID = W_pos.shape[1]

    info = plsc.get_sparse_core_info()
    sc_gather = _make_sc_gather(B, K, F, C, info.num_cores, info.num_subcores)
    td_g, mem_g = sc_gather(idx.astype(jnp.int32),
                            nns.astype(jnp.int32).reshape(-1),
                            train_dates, mem)

    BB = 1024
    grid = B // BB
    tc = pl.pallas_call(
        functools.partial(_tc_body, K, HID),
        grid=(grid,),
        in_specs=[
            pl.BlockSpec((BB, F), lambda i: (i, 0)),
            pl.BlockSpec((K, BB, F), lambda i: (0, i, 0)),
            pl.BlockSpec((K, BB, C), lambda i: (0, i, 0)),
            pl.BlockSpec((F, HID), lambda i: (0, 0)),
            pl.BlockSpec((1, HID), lambda i: (0, 0)),
            pl.BlockSpec((F, HID * C), lambda i: (0, 0)),
            pl.BlockSpec((1, HID * C), lambda i: (0, 0)),
            pl.BlockSpec((C, C), lambda i: (0, 0)),
            pl.BlockSpec((1, C), lambda i: (0, 0)),
        ],
        out_specs=pl.BlockSpec((BB, C), lambda i: (i, 0)),
        out_shape=jax.ShapeDtypeStruct((B, C), jnp.float32),
    )
    return tc(date, td_g, mem_g, W_pos, b_pos.reshape(1, HID), W_field,
              b_field.reshape(1, HID * C), W_adapt, b_adapt.reshape(1, C))
